# baseline (speedup 1.0000x reference)
import functools

import jax
import jax.numpy as jnp
from jax import lax
from jax.experimental import pallas as pl
from jax.experimental.pallas import tpu as pltpu
from jax.experimental.pallas import tpu_sc as plsc

L = 16
NC, NS = 2, 16
NW = NC * NS
BS = 16384
K, E, D = 16, 4, 64
B = 128
BLKS_PER_W = BS // (NW * B)
NB16 = B // L

_mesh = plsc.VectorSubcoreMesh(core_axis_name="c", subcore_axis_name="s")


@functools.partial(
    pl.kernel,
    out_type=(
        jax.ShapeDtypeStruct((D, BS), jnp.float32),
        jax.ShapeDtypeStruct((K, D, BS), jnp.float32),
    ),
    mesh=_mesh,
    compiler_params=pltpu.CompilerParams(needs_layout_passes=False),
    scratch_types=[
        pltpu.VMEM((2, E, D, B), jnp.float32),
        pltpu.VMEM((2, D, B), jnp.float32),
        pltpu.VMEM((2, D, B), jnp.float32),
        pltpu.VMEM((K, E, 1, B), jnp.int32),
        pltpu.VMEM((D, B), jnp.float32),
        pltpu.VMEM((D, B), jnp.float32),
        pltpu.SemaphoreType.DMA((2,)),
        pltpu.SemaphoreType.DMA((2,)),
        pltpu.SemaphoreType.DMA,
        pltpu.SemaphoreType.DMA,
        pltpu.SemaphoreType.DMA,
    ],
)
def _sc_agg(edge_hbm, ent_hbm, self_hbm, mask_hbm, sv_hbm, nbr_hbm,
            edge_v, ent_v, nbr_v, mask_v, self_v, sv_v,
            in_sem, out_sem, m_sem, f_sem, o_sem):
    wid = lax.axis_index("s") * NC + lax.axis_index("c")
    col0 = wid * (BLKS_PER_W * B)

    def start_mask(b0):
        pltpu.async_copy(mask_hbm.at[0, :, :, :, pl.ds(b0, B)], mask_v, m_sem)

    def wait_mask():
        pltpu.make_async_copy(mask_hbm.at[0, :, :, :, pl.ds(0, B)],
                              mask_v, m_sem).wait()

    def start_in(k, slot, b0):
        pltpu.async_copy(edge_hbm.at[k, :, :, pl.ds(b0, B)],
                         edge_v.at[slot], in_sem.at[slot])
        pltpu.async_copy(ent_hbm.at[k, :, pl.ds(b0, B)],
                         ent_v.at[slot], in_sem.at[slot])

    def wait_in(k, slot, b0):
        pltpu.make_async_copy(edge_hbm.at[k, :, :, pl.ds(b0, B)],
                              edge_v.at[slot], in_sem.at[slot]).wait()
        pltpu.make_async_copy(ent_hbm.at[k, :, pl.ds(b0, B)],
                              ent_v.at[slot], in_sem.at[slot]).wait()

    def wait_out(slot):
        pltpu.make_async_copy(nbr_v.at[slot], nbr_hbm.at[0, :, pl.ds(0, B)],
                              out_sem.at[slot]).wait()

    start_mask(col0)
    start_in(0, 0, col0)

    @pl.loop(0, BLKS_PER_W)
    def _blk(blk):
        b0 = col0 + blk * B

        pltpu.async_copy(self_hbm.at[:, pl.ds(b0, B)], self_v, f_sem)

        @pl.loop(0, D, unroll=4)
        def _z(d):
            for g in range(NB16):
                sv_v[d, pl.ds(g * L, L)] = jnp.zeros((L,), jnp.float32)

        wait_mask()

        @pl.loop(0, K // 2)
        def _kk(kk):
            for half in range(2):
                k = 2 * kk + half
                if half == 0:
                    start_in(k + 1, 1, b0)
                else:
                    @pl.when(kk < K // 2 - 1)
                    def _():
                        start_in(k + 1, 0, b0)

                    @pl.when((kk == K // 2 - 1) & (blk < BLKS_PER_W - 1))
                    def _():
                        start_in(0, 0, b0 + B)

                wait_in(k, half, b0)

                @pl.when(kk >= 1)
                def _():
                    wait_out(half)

                for g in range(NB16):
                    bb = g * L
                    ms = [mask_v[k, e, 0, pl.ds(bb, L)].astype(jnp.float32)
                          for e in range(E)]
                    cnt = (ms[0] + ms[1]) + (ms[2] + ms[3])
                    inv = 0.5 / jnp.maximum(cnt, 1.0)
                    cs = [m * inv for m in ms]

                    @plsc.parallel_loop(0, D, unroll=8)
                    def _d(d, half=half, bb=bb, cs=cs):
                        ev = [edge_v[half, e, d, pl.ds(bb, L)] for e in range(E)]
                        p01 = cs[0] * ev[0] + cs[1] * ev[1]
                        p23 = cs[2] * ev[2] + cs[3] * ev[3]
                        a = (ent_v[half, d, pl.ds(bb, L)] + p01) + p23
                        nbr_v[half, d, pl.ds(bb, L)] = a
                        sv_v[d, pl.ds(bb, L)] = sv_v[d, pl.ds(bb, L)] + a

                pltpu.async_copy(nbr_v.at[half],
                                 nbr_hbm.at[k, :, pl.ds(b0, B)],
                                 out_sem.at[half])

                if half == 1:
                    @pl.when((kk == K // 2 - 1) & (blk < BLKS_PER_W - 1))
                    def _():
                        start_mask(b0 + B)

        pltpu.make_async_copy(self_hbm.at[:, pl.ds(b0, B)],
                              self_v, f_sem).wait()

        @pl.loop(0, D, unroll=4)
        def _f(d):
            for g in range(NB16):
                sv_v[d, pl.ds(g * L, L)] = (
                    self_v[d, pl.ds(g * L, L)]
                    + (0.5 / K) * sv_v[d, pl.ds(g * L, L)])

        pltpu.async_copy(sv_v, sv_hbm.at[:, pl.ds(b0, B)], o_sem)
        wait_out(0)
        wait_out(1)
        pltpu.make_async_copy(sv_v, sv_hbm.at[:, pl.ds(b0, B)], o_sem).wait()


def kernel(self_vectors, neighbor_entity_vectors, neighbor_edge_vectors, masks):
    edge_t = jnp.transpose(neighbor_edge_vectors, (1, 2, 3, 4, 0))[0]
    ent_t = jnp.transpose(neighbor_entity_vectors, (1, 2, 3, 0))[0]
    self_t = self_vectors.T
    mask_t = jnp.transpose(masks, (1, 2, 3, 4, 0))
    sv_t, nbr_t = _sc_agg(edge_t, ent_t, self_t, mask_t)
    sv = sv_t.T.reshape(BS, 1, D)
    nbr = jnp.transpose(nbr_t, (2, 0, 1)).reshape(BS, 1, K, D)
    return sv, nbr
